# Initial kernel scaffold; baseline (speedup 1.0000x reference)
import jax
import jax.numpy as jnp
from jax.experimental import pallas as pl


def kernel(x, edge_index, batch, W1, b1, W2, b2, gamma, beta, fc_w, fc_b):
    raise NotImplementedError("write your pallas kernel here")


# speedup vs baseline: 4.4356x; 4.4356x over previous
import functools

import jax
import jax.numpy as jnp
from jax import lax
from jax.experimental import pallas as pl
from jax.experimental.pallas import tpu as pltpu
from jax.experimental.pallas import tpu_sc as plsc

NC = 2
NS = 16


def _sc_aggregate(h, src, dst, zrows):
    N, D = h.shape
    E = src.shape[0]
    NW = NC * NS
    EPW = E // NW
    CH = 80
    NCHUNK = EPW // CH
    RPT = (N // NS) // 8 * 8
    REM = N - RPT * NS

    mesh = plsc.VectorSubcoreMesh(core_axis_name="c", subcore_axis_name="s")

    @functools.partial(
        pl.kernel,
        out_type=jax.ShapeDtypeStruct((NC, N, D), jnp.float32),
        mesh=mesh,
        scratch_types=[
            pltpu.VMEM((CH,), jnp.int32),
            pltpu.VMEM((CH,), jnp.int32),
            pltpu.VMEM((CH, D), jnp.float32),
            pltpu.VMEM_SHARED((N, D), jnp.float32),
        ],
    )
    def agg_kernel(h_hbm, src_hbm, dst_hbm, z_hbm, out_hbm, si, di, rows, acc):
        c = lax.axis_index("c")
        s = lax.axis_index("s")
        wid = c * NS + s
        base = wid * EPW

        pltpu.sync_copy(z_hbm.at[pl.ds(0, RPT)], acc.at[pl.ds(s * RPT, RPT)])

        @pl.when(s == NS - 1)
        def _():
            pltpu.sync_copy(z_hbm.at[pl.ds(0, REM)],
                            acc.at[pl.ds(NS * RPT, REM)])

        plsc.subcore_barrier()

        @pl.loop(0, NCHUNK)
        def _(j):
            off = base + j * CH
            pltpu.sync_copy(src_hbm.at[pl.ds(off, CH)], si)
            pltpu.sync_copy(dst_hbm.at[pl.ds(off, CH)], di)
            pltpu.sync_copy(h_hbm.at[si], rows)
            pltpu.sync_copy(rows, acc.at[di], add=True)

        plsc.subcore_barrier()
        pltpu.sync_copy(acc.at[pl.ds(s * RPT, RPT)],
                        out_hbm.at[c, pl.ds(s * RPT, RPT)])

        @pl.when(s == NS - 1)
        def _():
            pltpu.sync_copy(acc.at[pl.ds(NS * RPT, REM)],
                            out_hbm.at[c, pl.ds(NS * RPT, REM)])

    return agg_kernel(h, src, dst, zrows)


def _tc_layer(h, agg, W1, b1, W2, b2, gamma, beta):
    N, D = h.shape

    def body(h_ref, a_ref, w1_ref, b1_ref, w2_ref, b2_ref, g_ref, bt_ref, o_ref):
        m = h_ref[...] + a_ref[0] + a_ref[1]
        t = jnp.dot(m.astype(jnp.bfloat16), w1_ref[...].astype(jnp.bfloat16),
                    preferred_element_type=jnp.float32)
        t = jnp.maximum(t + b1_ref[...], 0.0)
        u = jnp.dot(t.astype(jnp.bfloat16), w2_ref[...].astype(jnp.bfloat16),
                    preferred_element_type=jnp.float32)
        u = jnp.maximum(u + b2_ref[...], 0.0)
        mu = jnp.mean(u, axis=0, keepdims=True)
        d = u - mu
        var = jnp.mean(d * d, axis=0, keepdims=True)
        o_ref[...] = d * lax.rsqrt(var + 1e-5) * g_ref[...] + bt_ref[...]

    return pl.pallas_call(
        body,
        out_shape=jax.ShapeDtypeStruct((N, D), jnp.float32),
    )(h, agg, W1, b1.reshape(1, D), W2, b2.reshape(1, D),
      gamma.reshape(1, D), beta.reshape(1, D))


def _tc_pool(h, batch, G, fc_w, fc_b):
    N, D = h.shape

    def body(h_ref, b_ref, w_ref, bias_ref, o_ref):
        bvec = b_ref[...]
        gids = lax.broadcasted_iota(jnp.int32, (1, G), 1)
        onehot = (bvec == gids).astype(jnp.float32)
        cnt = jnp.sum(onehot, axis=0, keepdims=True)
        w = onehot * (1.0 / jnp.maximum(cnt, 1.0))
        pooled = lax.dot_general(w, h_ref[...], (((0,), (0,)), ((), ())),
                                 preferred_element_type=jnp.float32,
                                 precision=lax.Precision.HIGHEST)
        z = jnp.dot(pooled.astype(jnp.bfloat16), w_ref[...].astype(jnp.bfloat16),
                    preferred_element_type=jnp.float32)
        o_ref[...] = jnp.tanh(z + bias_ref[...])

    return pl.pallas_call(
        body,
        out_shape=jax.ShapeDtypeStruct((G, D), jnp.float32),
    )(h, batch.reshape(N, 1), fc_w, fc_b.reshape(1, D))


def kernel(x, edge_index, batch, W1, b1, W2, b2, gamma, beta, fc_w, fc_b):
    N, D = x.shape
    G = 64
    zrows = jnp.zeros(((N // NS) // 8 * 8, D), dtype=jnp.float32)
    src = edge_index[0]
    dst = edge_index[1]
    h = x
    for i in range(5):
        agg = _sc_aggregate(h, src, dst, zrows)
        h = _tc_layer(h, agg, W1[i], b1[i], W2[i], b2[i], gamma[i], beta[i])
    return _tc_pool(h, batch, G, fc_w, fc_b)
